# baseline (speedup 1.0000x reference)
import functools

import jax
import jax.numpy as jnp
from jax import lax
from jax.experimental import pallas as pl
from jax.experimental.pallas import tpu as pltpu
from jax.experimental.pallas import tpu_sc as plsc

_N_CODES = 1024
_DIM = 64
_BETA = 0.25
_T = 1024


def _vq_block(z_ref, emb_ref, idx_ref, dsum_ref, pad_ref):
    i = pl.program_id(0)
    z = z_ref[...]
    emb = emb_ref[...]
    znorm = jnp.sum(z * z, axis=1, keepdims=True)
    enorm = jnp.sum(emb * emb, axis=1)
    scores = jax.lax.dot_general(
        z, emb, (((1,), (1,)), ((), ())),
        preferred_element_type=jnp.float32,
        precision=jax.lax.Precision.DEFAULT)
    dist = (znorm - 2.0 * scores) + enorm[None, :]
    m = jnp.min(dist, axis=1, keepdims=True)
    eqb = jnp.where(dist == m, 1.0, 0.0).astype(jnp.bfloat16)
    colsel = jax.lax.broadcasted_iota(jnp.int32, (_N_CODES, 3), 1)
    code = jax.lax.broadcasted_iota(jnp.int32, (_N_CODES, 3), 0)
    li = jnp.where(colsel == 0, code >> 5,
                   jnp.where(colsel == 1, code & 31, 1)).astype(jnp.bfloat16)
    acc = jax.lax.dot_general(
        eqb, li, (((1,), (0,)), ((), ())),
        preferred_element_type=jnp.float32,
        precision=jax.lax.Precision.DEFAULT)
    idxf = acc[:, 0] * 32.0 + acc[:, 1]
    cnt = acc[:, 2]
    idx_ref[...] = idxf.astype(jnp.int32).reshape(_T // 128, 128)

    @pl.when(jnp.sum(cnt) > _T + 0.5)
    def _ties():
        lane = jax.lax.broadcasted_iota(jnp.int32, dist.shape, 1)
        idxs = jnp.min(jnp.where(dist == m, lane, _N_CODES), axis=1)
        idx_ref[...] = idxs.reshape(_T // 128, 128)

    @pl.when(i == 0)
    def _init():
        dsum_ref[...] = jnp.zeros_like(dsum_ref)
        pad_ref[...] = jnp.concatenate([emb, jnp.zeros_like(emb)], axis=1)

    dsum_ref[...] += jnp.sum(m).reshape(1, 1)


def _tc_stage(z_flat, embedding, phase, n_phases):
    n_tok = z_flat.shape[0]
    nb = n_tok // _T // n_phases
    off = phase * nb
    idx2d, dsum, table128 = pl.pallas_call(
        _vq_block,
        grid=(nb,),
        in_specs=[
            pl.BlockSpec((_T, _DIM), lambda i: (i + off, 0)),
            pl.BlockSpec((_N_CODES, _DIM), lambda i: (0, 0)),
        ],
        out_specs=[
            pl.BlockSpec((_T // 128, 128), lambda i: (i, 0)),
            pl.BlockSpec((1, 1), lambda i: (0, 0)),
            pl.BlockSpec((_N_CODES, 128), lambda i: (0, 0)),
        ],
        out_shape=[
            jax.ShapeDtypeStruct((n_tok // n_phases // 128, 128), jnp.int32),
            jax.ShapeDtypeStruct((1, 1), jnp.float32),
            jax.ShapeDtypeStruct((_N_CODES, 128), jnp.float32),
        ],
    )(z_flat, embedding)
    return idx2d, dsum, table128


def _sc_gather(table128, idx_flat):
    info = plsc.get_sparse_core_info()
    nw = info.num_cores * info.num_subcores
    b = idx_flat.shape[0]
    b_per_w = b // nw
    chunk = 512
    n_chunks = b_per_w // chunk
    mesh = plsc.VectorSubcoreMesh(core_axis_name="c", subcore_axis_name="s")

    @functools.partial(
        pl.kernel,
        out_type=jax.ShapeDtypeStruct((b, 128), jnp.float32),
        mesh=mesh,
        scratch_types=[
            pltpu.VMEM((chunk,), jnp.int32),
            pltpu.VMEM((chunk, 128), jnp.float32),
            pltpu.SemaphoreType.DMA,
        ],
    )
    def k(table_hbm, idx_hbm, out_hbm, idx_v, rows_v, sem):
        wid = lax.axis_index("s") * info.num_cores + lax.axis_index("c")
        base = wid * b_per_w
        for c in range(n_chunks):
            lo = base + c * chunk
            pltpu.sync_copy(idx_hbm.at[pl.ds(lo, chunk)], idx_v)
            pltpu.async_copy(table_hbm.at[idx_v], rows_v, sem).wait()
            pltpu.sync_copy(rows_v, out_hbm.at[pl.ds(lo, chunk)])

    return k(table128, idx_flat)


def kernel(z, embedding):
    n_tok = z.shape[0] * z.shape[1]
    half = n_tok // 2
    z_flat = z.reshape(n_tok, _DIM)
    idx2d_a, dsum_a, table128 = _tc_stage(z_flat, embedding, 0, 2)
    idx_a = idx2d_a.reshape(half)
    zq_pad_a = _sc_gather(table128, idx_a)
    idx2d_b, dsum_b, _ = _tc_stage(z_flat, embedding, 1, 2)
    idx_b = idx2d_b.reshape(half)
    zq_pad_b = _sc_gather(table128, idx_b)
    idx = jnp.concatenate([idx_a, idx_b])
    zq = jnp.concatenate(
        [zq_pad_a[:, :_DIM], zq_pad_b[:, :_DIM]], axis=0).reshape(z.shape)
    vq_loss = _BETA * ((dsum_a[0, 0] + dsum_b[0, 0]) / (n_tok * _DIM))
    return zq, idx, vq_loss


# speedup vs baseline: 1.0783x; 1.0783x over previous
import functools

import jax
import jax.numpy as jnp
from jax import lax
from jax.experimental import pallas as pl
from jax.experimental.pallas import tpu as pltpu
from jax.experimental.pallas import tpu_sc as plsc

_N_CODES = 1024
_DIM = 64
_BETA = 0.25
_T = 2048


def _vq_block(z_ref, emb_ref, idx_ref, dsum_ref, pad_ref):
    i = pl.program_id(0)
    z = z_ref[...]
    emb = emb_ref[...]
    znorm = jnp.sum(z * z, axis=1, keepdims=True)
    enorm = jnp.sum(emb * emb, axis=1)
    scores = jax.lax.dot_general(
        z, emb, (((1,), (1,)), ((), ())),
        preferred_element_type=jnp.float32,
        precision=jax.lax.Precision.DEFAULT)
    dist = (znorm - 2.0 * scores) + enorm[None, :]
    m = jnp.min(dist, axis=1, keepdims=True)
    eqb = jnp.where(dist == m, 1.0, 0.0).astype(jnp.bfloat16)
    colsel = jax.lax.broadcasted_iota(jnp.int32, (_N_CODES, 3), 1)
    code = jax.lax.broadcasted_iota(jnp.int32, (_N_CODES, 3), 0)
    li = jnp.where(colsel == 0, code >> 5,
                   jnp.where(colsel == 1, code & 31, 1)).astype(jnp.bfloat16)
    acc = jax.lax.dot_general(
        eqb, li, (((1,), (0,)), ((), ())),
        preferred_element_type=jnp.float32,
        precision=jax.lax.Precision.DEFAULT)
    idxf = acc[:, 0] * 32.0 + acc[:, 1]
    cnt = acc[:, 2]
    idx_ref[...] = idxf.astype(jnp.int32).reshape(_T // 128, 128)

    @pl.when(jnp.sum(cnt) > _T + 0.5)
    def _ties():
        lane = jax.lax.broadcasted_iota(jnp.int32, dist.shape, 1)
        idxs = jnp.min(jnp.where(dist == m, lane, _N_CODES), axis=1)
        idx_ref[...] = idxs.reshape(_T // 128, 128)

    @pl.when(i == 0)
    def _init():
        dsum_ref[...] = jnp.zeros_like(dsum_ref)
        pad_ref[...] = jnp.concatenate([emb, jnp.zeros_like(emb)], axis=1)

    dsum_ref[...] += jnp.sum(m).reshape(1, 1)


def _tc_stage(z_flat, embedding, phase, n_phases):
    n_tok = z_flat.shape[0]
    nb = n_tok // _T // n_phases
    off = phase * nb
    idx2d, dsum, table128 = pl.pallas_call(
        _vq_block,
        grid=(nb,),
        in_specs=[
            pl.BlockSpec((_T, _DIM), lambda i: (i + off, 0)),
            pl.BlockSpec((_N_CODES, _DIM), lambda i: (0, 0)),
        ],
        out_specs=[
            pl.BlockSpec((_T // 128, 128), lambda i: (i, 0)),
            pl.BlockSpec((1, 1), lambda i: (0, 0)),
            pl.BlockSpec((_N_CODES, 128), lambda i: (0, 0)),
        ],
        out_shape=[
            jax.ShapeDtypeStruct((n_tok // n_phases // 128, 128), jnp.int32),
            jax.ShapeDtypeStruct((1, 1), jnp.float32),
            jax.ShapeDtypeStruct((_N_CODES, 128), jnp.float32),
        ],
    )(z_flat, embedding)
    return idx2d, dsum, table128


def _sc_gather(table128, idx_flat):
    info = plsc.get_sparse_core_info()
    nw = info.num_cores * info.num_subcores
    b = idx_flat.shape[0]
    b_per_w = b // nw
    chunk = 512
    n_chunks = b_per_w // chunk
    mesh = plsc.VectorSubcoreMesh(core_axis_name="c", subcore_axis_name="s")

    @functools.partial(
        pl.kernel,
        out_type=jax.ShapeDtypeStruct((b, 128), jnp.float32),
        mesh=mesh,
        scratch_types=[
            pltpu.VMEM((chunk,), jnp.int32),
            pltpu.VMEM((chunk, 128), jnp.float32),
            pltpu.SemaphoreType.DMA,
        ],
    )
    def k(table_hbm, idx_hbm, out_hbm, idx_v, rows_v, sem):
        wid = lax.axis_index("s") * info.num_cores + lax.axis_index("c")
        base = wid * b_per_w
        for c in range(n_chunks):
            lo = base + c * chunk
            pltpu.sync_copy(idx_hbm.at[pl.ds(lo, chunk)], idx_v)
            pltpu.async_copy(table_hbm.at[idx_v], rows_v, sem).wait()
            pltpu.sync_copy(rows_v, out_hbm.at[pl.ds(lo, chunk)])

    return k(table128, idx_flat)


def kernel(z, embedding):
    n_tok = z.shape[0] * z.shape[1]
    z_flat = z.reshape(n_tok, _DIM)
    idx2d, dsum, table128 = _tc_stage(z_flat, embedding, 0, 1)
    idx = idx2d.reshape(n_tok)
    zq_pad = _sc_gather(table128, idx)
    zq = zq_pad[:, :_DIM].reshape(z.shape)
    vq_loss = _BETA * (dsum[0, 0] / (n_tok * _DIM))
    return zq, idx, vq_loss
